# baseline (speedup 1.0000x reference)
import jax
import jax.numpy as jnp
from jax import lax
from jax.experimental import pallas as pl
from jax.experimental.pallas import tpu as pltpu
from jax.experimental.pallas import tpu_sc as plsc

VOCAB = 1000000
EMBED = 64
BATCH = 4096
HIST = 200

NUM_CORES = 2
NUM_SUBCORES = 16
NUM_WORKERS = NUM_CORES * NUM_SUBCORES
ROWS_PER_WORKER = BATCH // NUM_WORKERS
CHUNK_ROWS = 2
CHUNK_IDS = CHUNK_ROWS * HIST
NUM_CHUNKS = ROWS_PER_WORKER // CHUNK_ROWS
LANES = 16
VPR = EMBED // LANES
PACK = 2 * EMBED

_splits = []
_off = 0
while _off < CHUNK_IDS:
    _sz = min(128, CHUNK_IDS - _off)
    _splits.append((_off, _sz))
    _off += _sz
GATHER_SPLITS = tuple(_splits)

_IDX_PAD = 16


def _pool_kernel(ids_hbm, packed_hbm, out_hbm,
                 idx_v, pidx_v, rows_v, out_loc, sem0, sem1):
    wid = lax.axis_index("s") * NUM_CORES + lax.axis_index("c")
    ids_base = wid * ROWS_PER_WORKER * HIST
    sems = (sem0, sem1)

    def stage(c, buf):
        pltpu.sync_copy(
            ids_hbm.at[pl.ds(ids_base + c * CHUNK_IDS, CHUNK_IDS)],
            idx_v.at[buf, pl.ds(0, CHUNK_IDS)],
        )

        @pl.loop(0, CHUNK_IDS // LANES)
        def _shift(k):
            v = idx_v[buf, pl.ds(k * LANES, LANES)]
            pidx_v[buf, pl.ds(k * LANES, LANES)] = (
                lax.shift_right_logical(v, 1))

        for off, sz in GATHER_SPLITS:
            pltpu.async_copy(
                packed_hbm.at[pidx_v.at[buf, pl.ds(off, sz)]],
                rows_v.at[buf, pl.ds(off, sz)],
                sems[buf],
            )

    def drain(buf):
        pltpu.make_async_copy(
            packed_hbm.at[pl.ds(0, CHUNK_IDS)], rows_v.at[buf], sems[buf]
        ).wait()

    def pool(c, buf):
        for r in range(CHUNK_ROWS):
            zeros = tuple(jnp.zeros((LANES,), jnp.float32)
                          for _ in range(VPR))

            @pl.loop(0, HIST // 8, init_carry=zeros)
            def accs(k, acc):
                idv = idx_v[buf, pl.ds(r * HIST + k * 8, LANES)]
                pf = (idv & 1).astype(jnp.float32)
                for l in range(8):
                    j = r * HIST + k * 8 + l
                    ps = jnp.take_along_axis(
                        pf, jnp.full((LANES,), l, jnp.int32), axis=0)
                    acc = tuple(
                        acc[v]
                        + rows_v[buf, j, pl.ds(v * LANES, LANES)]
                        + ps * (rows_v[buf, j, pl.ds(EMBED + v * LANES, LANES)]
                                - rows_v[buf, j, pl.ds(v * LANES, LANES)])
                        for v in range(VPR)
                    )
                return acc

            for v in range(VPR):
                out_loc[c, pl.ds(r * EMBED + v * LANES, LANES)] = (
                    accs[v] * (1.0 / HIST))

    stage(0, 0)

    @pl.loop(0, NUM_CHUNKS, step=2)
    def _main(cc):
        for b in range(2):
            c = cc + b
            if b == 0:
                stage(cc + 1, 1)
            else:
                @pl.when(cc < NUM_CHUNKS - 2)
                def _():
                    stage(cc + 2, 0)
            drain(b)
            pool(c, b)

    pltpu.sync_copy(
        out_loc,
        out_hbm.at[pl.ds(wid * (ROWS_PER_WORKER // 2), ROWS_PER_WORKER // 2)],
    )


@jax.jit
def _pooled_means(ids_flat, packed_table):
    mesh = plsc.VectorSubcoreMesh(core_axis_name="c", subcore_axis_name="s")
    return pl.kernel(
        _pool_kernel,
        out_type=jax.ShapeDtypeStruct((BATCH // 2, PACK), jnp.float32),
        mesh=mesh,
        compiler_params=pltpu.CompilerParams(use_tc_tiling_on_sc=False),
        scratch_types=[
            pltpu.VMEM((2, CHUNK_IDS + _IDX_PAD), jnp.int32),
            pltpu.VMEM((2, CHUNK_IDS), jnp.int32),
            pltpu.VMEM((2, CHUNK_IDS, PACK), jnp.float32),
            pltpu.VMEM((ROWS_PER_WORKER // 2, PACK), jnp.float32),
            pltpu.SemaphoreType.DMA,
            pltpu.SemaphoreType.DMA,
        ],
    )(ids_flat, packed_table)


def _proj_kernel(x_ref, w_ref, b_ref, o_ref):
    o_ref[...] = (
        lax.dot_general(
            x_ref[...], w_ref[...],
            (((1,), (1,)), ((), ())),
            preferred_element_type=jnp.float32,
        )
        + b_ref[...]
    )


@jax.jit
def _project(pooled, W, b2d):
    return pl.pallas_call(
        _proj_kernel,
        out_shape=jax.ShapeDtypeStruct((BATCH, EMBED), jnp.float32),
    )(pooled, W, b2d)


def kernel(input_ids, emb_table, W, b):
    ids_flat = input_ids.reshape(-1).astype(jnp.int32)
    packed_table = emb_table.reshape(VOCAB // 2, PACK)
    pooled_packed = _pooled_means(ids_flat, packed_table)
    pooled = pooled_packed.reshape(BATCH, EMBED)
    return _project(pooled, W, b.reshape(1, EMBED))
